# baseline (speedup 1.0000x reference)
import functools

import jax
import jax.numpy as jnp
import numpy as np
from jax import lax
from jax.experimental import pallas as pl
from jax.experimental.pallas import tpu as pltpu
from jax.experimental.pallas import tpu_sc as plsc

B = 16384
F = 26
D = 16
FD = F * D
FIELD_DIM = 38462
TOTAL_ROWS = F * FIELD_DIM
EPS = 1e-5
N_IDX = B * F

NC = 2
NS = 16
NW = NC * NS
PER_W = N_IDX // NW
SAMP_W = PER_W // F
CHUNK = 1024
N_CHUNKS = PER_W // CHUNK
LIN_ROWS = (TOTAL_ROWS + 15) // 16 + 1

TAB_ROWS = 1000016
OUT128 = TAB_ROWS * D // 128
TCOLS = 999424
W_COLS = TCOLS // NW
TCHUNK = 512
N_TCH = W_COLS // TCHUNK
TAIL_OUT0 = TCOLS * D // 128
TAIL128 = OUT128 - TAIL_OUT0

_OFFSETS = np.arange(F, dtype=np.int32) * FIELD_DIM


@functools.cache
def _sc_transpose_fn():
    mesh = plsc.VectorSubcoreMesh(core_axis_name="c", subcore_axis_name="s")

    @functools.partial(
        pl.kernel,
        mesh=mesh,
        out_type=jax.ShapeDtypeStruct((OUT128, 128), jnp.float32),
        scratch_types=[
            pltpu.VMEM((128, 128), jnp.float32),
            pltpu.VMEM((2, TCHUNK // 8, 128), jnp.float32),
            pltpu.VMEM((TAIL128, 128), jnp.float32),
            pltpu.SemaphoreType.DMA,
            pltpu.SemaphoreType.DMA,
        ],
        compiler_params=pltpu.CompilerParams(use_tc_tiling_on_sc=True,
                                             needs_layout_passes=False),
    )
    def sc_transpose(embt_hbm, tail_hbm, out_hbm, in_v, out_v, tail_v,
                     sem_i, sem_o):
        wid = lax.axis_index("s") * NC + lax.axis_index("c")
        c0w = wid * W_COLS
        lanes = lax.iota(jnp.int32, 16)

        def start_in(g):
            off = pl.multiple_of(c0w + g * TCHUNK, TCHUNK)
            for t in range(8):
                tr, tc = t // 4, t % 4
                pltpu.async_copy(
                    embt_hbm.at[pl.ds(tr * 8, 8), pl.ds(off + tc * 128, 128)],
                    in_v.at[pl.ds((g % 2) * 64 + t * 8, 8)], sem_i)

        def wait_in():
            for _ in range(8):
                pltpu.make_async_copy(
                    embt_hbm.at[pl.ds(0, 8), pl.ds(0, 128)],
                    in_v.at[pl.ds(0, 8)], sem_i).wait()

        def start_out(g):
            oof = pl.multiple_of((c0w + g * TCHUNK) // 8, TCHUNK // 8)
            pltpu.async_copy(out_v.at[g % 2],
                             out_hbm.at[pl.ds(oof, TCHUNK // 8)], sem_o)

        def wait_out():
            pltpu.make_async_copy(out_v.at[0],
                                  out_hbm.at[pl.ds(0, TCHUNK // 8)],
                                  sem_o).wait()

        lane_row = (lanes // 8) * 32 + lanes % 8

        def compute(g):
            buf = g % 2
            row_d = lane_row + buf * 64

            @plsc.parallel_loop(0, TCHUNK // 8, unroll=8)
            def trans(jj):
                j0 = jj * 8
                rowv = row_d + (j0 // 128) * 8
                cm = j0 % 128
                for u in range(8):
                    colv = jnp.full((16,), cm + u, jnp.int32)
                    vals = plsc.load_gather(in_v, [rowv, colv])
                    out_v[buf, jj, pl.ds(u * 16, 16)] = vals

        start_in(0)

        def obody(g, carry):
            start_in(g + 1)
            wait_in()

            @pl.when(g >= 2)
            def _():
                wait_out()

            compute(g)
            start_out(g)
            return carry

        lax.fori_loop(0, N_TCH - 1, obody, 0)
        wait_in()
        wait_out()
        compute(N_TCH - 1)
        start_out(N_TCH - 1)
        wait_out()
        wait_out()

        @pl.when(wid == 0)
        def _():
            pltpu.sync_copy(tail_hbm, tail_v)
            pltpu.sync_copy(tail_v, out_hbm.at[pl.ds(TAIL_OUT0, TAIL128)])

    return sc_transpose


@functools.cache
def _sc_gather_fn():
    mesh = plsc.VectorSubcoreMesh(core_axis_name="c", subcore_axis_name="s")

    @functools.partial(
        pl.kernel,
        mesh=mesh,
        out_type=[
            jax.ShapeDtypeStruct((N_IDX, D), jnp.float32),
            jax.ShapeDtypeStruct((B,), jnp.float32),
        ],
        name="fm_sc_gather",
        scratch_types=[
            pltpu.VMEM((PER_W,), jnp.int32),
            pltpu.VMEM((PER_W,), jnp.int32),
            pltpu.VMEM((2, CHUNK, D), jnp.float32),
            pltpu.VMEM((2, CHUNK, D), jnp.float32),
            pltpu.VMEM((PER_W,), jnp.float32),
            pltpu.VMEM((SAMP_W,), jnp.float32),
            pltpu.SemaphoreType.DMA,
            pltpu.SemaphoreType.DMA,
        ],
        compiler_params=pltpu.CompilerParams(use_tc_tiling_on_sc=False,
                                             needs_layout_passes=False),
    )
    def sc_gather(idx_hbm, hi_hbm, emb_hbm, lin16_hbm, out_emb, out_lin,
                  idx_v, hi_v, rows_v, lin16_v, comp_v, lsum_v, sem_e, sem_l):
        wid = lax.axis_index("s") * NC + lax.axis_index("c")
        base = wid * PER_W
        pltpu.sync_copy(idx_hbm.at[wid], idx_v)
        pltpu.sync_copy(hi_hbm.at[wid], hi_v)
        lanes = lax.iota(jnp.int32, 16)

        def start(g):
            buf = g % 2
            isl = idx_v.at[pl.ds(g * CHUNK, CHUNK)]
            hsl = hi_v.at[pl.ds(g * CHUNK, CHUNK)]
            cpe = pltpu.async_copy(emb_hbm.at[isl], rows_v.at[buf], sem_e)
            cpl = pltpu.async_copy(lin16_hbm.at[hsl], lin16_v.at[buf], sem_l)
            return cpe, cpl

        pend = start(0)
        for g in range(N_CHUNKS):
            cpe, cpl = pend
            cpe.wait()
            cpl.wait()
            if g + 1 < N_CHUNKS:
                pend = start(g + 1)
            buf = g % 2
            bufv = jnp.full((16,), buf, dtype=jnp.int32)

            @plsc.parallel_loop(0, CHUNK // 16, unroll=4)
            def compact(k):
                rem = idx_v[pl.ds(g * CHUNK + k * 16, 16)] & 15
                row = lanes + k * 16
                vals = plsc.load_gather(lin16_v, [bufv, row, rem])
                comp_v[pl.ds(g * CHUNK + k * 16, 16)] = vals
            pltpu.sync_copy(rows_v.at[buf],
                            out_emb.at[pl.ds(base + g * CHUNK, CHUNK)])

        lane26 = lanes * F

        @plsc.parallel_loop(0, SAMP_W // 16, unroll=2)
        def sample_sum(g):
            pos0 = lane26 + g * (16 * F)
            acc = plsc.load_gather(comp_v, [pos0])
            for f in range(1, F):
                acc = acc + plsc.load_gather(comp_v, [pos0 + f])
            lsum_v[pl.ds(g * 16, 16)] = acc
        pltpu.sync_copy(lsum_v, out_lin.at[pl.ds(wid * SAMP_W, SAMP_W)])

    return sc_gather


BB1 = 2048
BB2 = 2048
BF = jnp.bfloat16
F32 = jnp.float32


def _split_dot(a, b_bf, prec_a=True):
    ah = a.astype(BF)
    al = (a - ah.astype(F32)).astype(BF)
    out = jnp.dot(ah, b_bf, preferred_element_type=F32)
    if prec_a:
        out = out + jnp.dot(al, b_bf, preferred_element_type=F32)
    return out


def _stats_kernel(emb_ref, et_ref, em_ref, g_ref, b_ref,
                  sum_ref, sq_ref, sc_ref, sh_ref):
    i = pl.program_id(0)
    e = emb_ref[...]
    s = jnp.sum(e, axis=0, keepdims=True)
    q = jnp.sum(e * e, axis=0, keepdims=True)

    @pl.when(i == 0)
    def _():
        sum_ref[...] = s
        sq_ref[...] = q

    @pl.when(i > 0)
    def _():
        sum_ref[...] += s
        sq_ref[...] += q

    @pl.when(i == B // BB1 - 1)
    def _():
        inv_n = 1.0 / (B * D)
        sf = _split_dot(sum_ref[...], et_ref[...]) * inv_n
        qf = _split_dot(sq_ref[...], et_ref[...]) * inv_n
        var = qf - sf * sf
        scale = g_ref[...] * lax.rsqrt(var + EPS)
        shift = b_ref[...] - sf * scale
        sc_ref[...] = _split_dot(scale, em_ref[...])
        sh_ref[...] = _split_dot(shift, em_ref[...])


def _main_kernel(emb_ref, lin_ref, sc_ref, sh_ref, e_ref, p_ref,
                 cw_ref, cb_ref, lb_ref, out_ref):
    em = e_ref[...]
    xn = emb_ref[...] * sc_ref[...] + sh_ref[...]
    cw = cw_ref[...]
    cwh = cw.astype(BF)
    cwl = (cw - cwh.astype(F32)).astype(BF)
    xh = xn.astype(BF)
    xl = (xn - xh.astype(F32)).astype(BF)
    z = (jnp.dot(xh, cwh, preferred_element_type=F32)
         + jnp.dot(xh, cwl, preferred_element_type=F32)
         + jnp.dot(xl, cwh, preferred_element_type=F32)) + cb_ref[...]
    w = 1.0 / (1.0 + jnp.exp(-z))
    w416 = _split_dot(w, em)
    ex = xn * w416
    sd = _split_dot(ex, p_ref[...])
    fm = 0.5 * (jnp.sum(sd * sd, axis=1) - jnp.sum(ex * ex, axis=1))
    out_ref[...] = 1.0 / (1.0 + jnp.exp(-(lin_ref[...] + lb_ref[0, 0] + fm)))


def kernel(x, emb_table, lin_w, lin_b, ctrl_w, ctrl_b, bn_gamma, bn_beta):
    offsets = jnp.asarray(_OFFSETS)
    idx = (x + offsets[None, :]).reshape(NW, PER_W)
    idx_hi = idx >> 4
    lin16 = jnp.pad(lin_w.reshape(-1), (0, LIN_ROWS * 16 - TOTAL_ROWS)
                    ).reshape(LIN_ROWS, 16)
    emb_t = jnp.transpose(emb_table)
    tail = jnp.pad(emb_table[TCOLS:], ((0, TAB_ROWS - TOTAL_ROWS), (0, 0)))
    tail128 = tail.reshape(TAIL128, 128)
    rm128 = _sc_transpose_fn()(emb_t, tail128)
    emb_rm = rm128.reshape(TAB_ROWS, D)

    gat_emb, lin_sum = _sc_gather_fn()(idx, idx_hi, emb_rm, lin16)
    emb2d = gat_emb.reshape(B, FD)

    jfd = jnp.arange(FD, dtype=jnp.int32)
    jf = jnp.arange(F, dtype=jnp.int32)
    jd = jnp.arange(D, dtype=jnp.int32)
    e_mat = (jfd[None, :] // D == jf[:, None]).astype(BF)
    et_mat = (jfd[:, None] // D == jf[None, :]).astype(BF)
    p_mat = (jfd[:, None] % D == jd[None, :]).astype(BF)

    full = lambda shape: pl.BlockSpec(shape, lambda i: tuple(0 for _ in shape))
    _, _, sc416, sh416 = pl.pallas_call(
        _stats_kernel,
        grid=(B // BB1,),
        in_specs=[pl.BlockSpec((BB1, FD), lambda i: (i, 0)),
                  full((FD, F)), full((F, FD)), full((1, F)), full((1, F))],
        out_specs=[
            pl.BlockSpec((1, FD), lambda i: (0, 0)),
            pl.BlockSpec((1, FD), lambda i: (0, 0)),
            pl.BlockSpec((1, FD), lambda i: (0, 0)),
            pl.BlockSpec((1, FD), lambda i: (0, 0)),
        ],
        out_shape=[
            jax.ShapeDtypeStruct((1, FD), jnp.float32),
            jax.ShapeDtypeStruct((1, FD), jnp.float32),
            jax.ShapeDtypeStruct((1, FD), jnp.float32),
            jax.ShapeDtypeStruct((1, FD), jnp.float32),
        ],
    )(emb2d, et_mat, e_mat, bn_gamma.reshape(1, F), bn_beta.reshape(1, F))

    out = pl.pallas_call(
        _main_kernel,
        grid=(B // BB2,),
        in_specs=[
            pl.BlockSpec((BB2, FD), lambda i: (i, 0)),
            pl.BlockSpec((BB2,), lambda i: (i,)),
            full((1, FD)),
            full((1, FD)),
            full((F, FD)),
            full((FD, D)),
            full((FD, F)),
            full((1, F)),
            full((1, 1)),
        ],
        out_specs=pl.BlockSpec((BB2,), lambda i: (i,)),
        out_shape=jax.ShapeDtypeStruct((B,), jnp.float32),
    )(emb2d, lin_sum, sc416, sh416, e_mat, p_mat, ctrl_w,
      ctrl_b.reshape(1, F), lin_b.reshape(1, 1))
    return out


# speedup vs baseline: 1.0169x; 1.0169x over previous
import functools

import jax
import jax.numpy as jnp
import numpy as np
from jax import lax
from jax.experimental import pallas as pl
from jax.experimental.pallas import tpu as pltpu
from jax.experimental.pallas import tpu_sc as plsc

B = 16384
F = 26
D = 16
FD = F * D
FIELD_DIM = 38462
TOTAL_ROWS = F * FIELD_DIM
EPS = 1e-5
N_IDX = B * F

NC = 2
NS = 16
NW = NC * NS
PER_W = N_IDX // NW
SAMP_W = PER_W // F
CHUNK = 1024
N_CHUNKS = PER_W // CHUNK
LIN_ROWS = (TOTAL_ROWS + 15) // 16 + 1

TAB_ROWS = 1000016
OUT128 = TAB_ROWS * D // 128
TCOLS = 999424
W_COLS = TCOLS // NW
TCHUNK = 512
N_TCH = W_COLS // TCHUNK
TAIL_OUT0 = TCOLS * D // 128
TAIL128 = OUT128 - TAIL_OUT0

_OFFSETS = np.arange(F, dtype=np.int32) * FIELD_DIM


@functools.cache
def _sc_transpose_fn():
    mesh = plsc.VectorSubcoreMesh(core_axis_name="c", subcore_axis_name="s")

    @functools.partial(
        pl.kernel,
        mesh=mesh,
        out_type=jax.ShapeDtypeStruct((OUT128, 128), jnp.float32),
        scratch_types=[
            pltpu.VMEM((2, 8, 8, 128), jnp.float32),
            pltpu.VMEM((2, TCHUNK // 8, 128), jnp.float32),
            pltpu.VMEM((TAIL128, 128), jnp.float32),
            pltpu.SemaphoreType.DMA,
            pltpu.SemaphoreType.DMA,
        ],
        compiler_params=pltpu.CompilerParams(use_tc_tiling_on_sc=True,
                                             needs_layout_passes=False),
    )
    def sc_transpose(embt_hbm, tail_hbm, out_hbm, in_v, out_v, tail_v,
                     sem_i, sem_o):
        wid = lax.axis_index("s") * NC + lax.axis_index("c")
        c0w = wid * W_COLS
        lanes = lax.iota(jnp.int32, 16)

        def start_in(g):
            off = pl.multiple_of(c0w + g * TCHUNK, TCHUNK)
            for t in range(8):
                tr, tc = t // 4, t % 4
                pltpu.async_copy(
                    embt_hbm.at[pl.ds(tr * 8, 8), pl.ds(off + tc * 128, 128)],
                    in_v.at[g % 2, t], sem_i)

        def wait_in():
            for _ in range(8):
                pltpu.make_async_copy(
                    embt_hbm.at[pl.ds(0, 8), pl.ds(0, 128)],
                    in_v.at[0, 0], sem_i).wait()

        def start_out(g):
            oof = pl.multiple_of((c0w + g * TCHUNK) // 8, TCHUNK // 8)
            pltpu.async_copy(out_v.at[g % 2],
                             out_hbm.at[pl.ds(oof, TCHUNK // 8)], sem_o)

        def wait_out():
            pltpu.make_async_copy(out_v.at[0],
                                  out_hbm.at[pl.ds(0, TCHUNK // 8)],
                                  sem_o).wait()

        lane_t4 = (lanes // 8) * 4
        lane_r = lanes % 8

        def compute(g):
            buf = g % 2
            bufv = jnp.full((16,), buf, jnp.int32)

            @plsc.parallel_loop(0, TCHUNK // 8, unroll=8)
            def trans(jj):
                j0 = jj * 8
                tvec = lane_t4 + j0 // 128
                cm = j0 % 128
                for u in range(8):
                    colv = jnp.full((16,), cm + u, jnp.int32)
                    vals = plsc.load_gather(in_v, [bufv, tvec, lane_r, colv])
                    out_v[buf, jj, pl.ds(u * 16, 16)] = vals

        start_in(0)

        def obody(g, carry):
            start_in(g + 1)
            wait_in()

            @pl.when(g >= 2)
            def _():
                wait_out()

            compute(g)
            start_out(g)
            return carry

        lax.fori_loop(0, N_TCH - 1, obody, 0)
        wait_in()
        wait_out()
        compute(N_TCH - 1)
        start_out(N_TCH - 1)
        wait_out()
        wait_out()

        @pl.when(wid == 0)
        def _():
            pltpu.sync_copy(tail_hbm, tail_v)
            pltpu.sync_copy(tail_v, out_hbm.at[pl.ds(TAIL_OUT0, TAIL128)])

    return sc_transpose


@functools.cache
def _sc_gather_fn():
    mesh = plsc.VectorSubcoreMesh(core_axis_name="c", subcore_axis_name="s")

    @functools.partial(
        pl.kernel,
        mesh=mesh,
        out_type=[
            jax.ShapeDtypeStruct((N_IDX, D), jnp.float32),
            jax.ShapeDtypeStruct((B,), jnp.float32),
        ],
        name="fm_sc_gather",
        scratch_types=[
            pltpu.VMEM((PER_W,), jnp.int32),
            pltpu.VMEM((PER_W,), jnp.int32),
            pltpu.VMEM((2, CHUNK, D), jnp.float32),
            pltpu.VMEM((2, CHUNK, D), jnp.float32),
            pltpu.VMEM((PER_W,), jnp.float32),
            pltpu.VMEM((SAMP_W,), jnp.float32),
            pltpu.SemaphoreType.DMA,
            pltpu.SemaphoreType.DMA,
        ],
        compiler_params=pltpu.CompilerParams(use_tc_tiling_on_sc=False,
                                             needs_layout_passes=False),
    )
    def sc_gather(idx_hbm, hi_hbm, emb_hbm, lin16_hbm, out_emb, out_lin,
                  idx_v, hi_v, rows_v, lin16_v, comp_v, lsum_v, sem_e, sem_l):
        wid = lax.axis_index("s") * NC + lax.axis_index("c")
        base = wid * PER_W
        pltpu.sync_copy(idx_hbm.at[wid], idx_v)
        pltpu.sync_copy(hi_hbm.at[wid], hi_v)
        lanes = lax.iota(jnp.int32, 16)

        def start(g):
            buf = g % 2
            isl = idx_v.at[pl.ds(g * CHUNK, CHUNK)]
            hsl = hi_v.at[pl.ds(g * CHUNK, CHUNK)]
            cpe = pltpu.async_copy(emb_hbm.at[isl], rows_v.at[buf], sem_e)
            cpl = pltpu.async_copy(lin16_hbm.at[hsl], lin16_v.at[buf], sem_l)
            return cpe, cpl

        pend = start(0)
        for g in range(N_CHUNKS):
            cpe, cpl = pend
            cpe.wait()
            cpl.wait()
            if g + 1 < N_CHUNKS:
                pend = start(g + 1)
            buf = g % 2
            bufv = jnp.full((16,), buf, dtype=jnp.int32)

            @plsc.parallel_loop(0, CHUNK // 16, unroll=4)
            def compact(k):
                rem = idx_v[pl.ds(g * CHUNK + k * 16, 16)] & 15
                row = lanes + k * 16
                vals = plsc.load_gather(lin16_v, [bufv, row, rem])
                comp_v[pl.ds(g * CHUNK + k * 16, 16)] = vals
            pltpu.sync_copy(rows_v.at[buf],
                            out_emb.at[pl.ds(base + g * CHUNK, CHUNK)])

        lane26 = lanes * F

        @plsc.parallel_loop(0, SAMP_W // 16, unroll=2)
        def sample_sum(g):
            pos0 = lane26 + g * (16 * F)
            acc = plsc.load_gather(comp_v, [pos0])
            for f in range(1, F):
                acc = acc + plsc.load_gather(comp_v, [pos0 + f])
            lsum_v[pl.ds(g * 16, 16)] = acc
        pltpu.sync_copy(lsum_v, out_lin.at[pl.ds(wid * SAMP_W, SAMP_W)])

    return sc_gather


BB1 = 2048
BB2 = 2048
BF = jnp.bfloat16
F32 = jnp.float32


def _split_dot(a, b_bf, prec_a=True):
    ah = a.astype(BF)
    al = (a - ah.astype(F32)).astype(BF)
    out = jnp.dot(ah, b_bf, preferred_element_type=F32)
    if prec_a:
        out = out + jnp.dot(al, b_bf, preferred_element_type=F32)
    return out


def _stats_kernel(emb_ref, et_ref, em_ref, g_ref, b_ref,
                  sum_ref, sq_ref, sc_ref, sh_ref):
    i = pl.program_id(0)
    e = emb_ref[...]
    s = jnp.sum(e, axis=0, keepdims=True)
    q = jnp.sum(e * e, axis=0, keepdims=True)

    @pl.when(i == 0)
    def _():
        sum_ref[...] = s
        sq_ref[...] = q

    @pl.when(i > 0)
    def _():
        sum_ref[...] += s
        sq_ref[...] += q

    @pl.when(i == B // BB1 - 1)
    def _():
        inv_n = 1.0 / (B * D)
        sf = _split_dot(sum_ref[...], et_ref[...]) * inv_n
        qf = _split_dot(sq_ref[...], et_ref[...]) * inv_n
        var = qf - sf * sf
        scale = g_ref[...] * lax.rsqrt(var + EPS)
        shift = b_ref[...] - sf * scale
        sc_ref[...] = _split_dot(scale, em_ref[...])
        sh_ref[...] = _split_dot(shift, em_ref[...])


def _main_kernel(emb_ref, lin_ref, sc_ref, sh_ref, e_ref, p_ref,
                 cw_ref, cb_ref, lb_ref, out_ref):
    em = e_ref[...]
    xn = emb_ref[...] * sc_ref[...] + sh_ref[...]
    cw = cw_ref[...]
    cwh = cw.astype(BF)
    cwl = (cw - cwh.astype(F32)).astype(BF)
    xh = xn.astype(BF)
    xl = (xn - xh.astype(F32)).astype(BF)
    z = (jnp.dot(xh, cwh, preferred_element_type=F32)
         + jnp.dot(xh, cwl, preferred_element_type=F32)
         + jnp.dot(xl, cwh, preferred_element_type=F32)) + cb_ref[...]
    w = 1.0 / (1.0 + jnp.exp(-z))
    w416 = _split_dot(w, em)
    ex = xn * w416
    sd = _split_dot(ex, p_ref[...])
    fm = 0.5 * (jnp.sum(sd * sd, axis=1) - jnp.sum(ex * ex, axis=1))
    out_ref[...] = 1.0 / (1.0 + jnp.exp(-(lin_ref[...] + lb_ref[0, 0] + fm)))


def kernel(x, emb_table, lin_w, lin_b, ctrl_w, ctrl_b, bn_gamma, bn_beta):
    offsets = jnp.asarray(_OFFSETS)
    idx = (x + offsets[None, :]).reshape(NW, PER_W)
    idx_hi = idx >> 4
    lin16 = jnp.pad(lin_w.reshape(-1), (0, LIN_ROWS * 16 - TOTAL_ROWS)
                    ).reshape(LIN_ROWS, 16)
    emb_t = jnp.transpose(emb_table)
    tail = jnp.pad(emb_table[TCOLS:], ((0, TAB_ROWS - TOTAL_ROWS), (0, 0)))
    tail128 = tail.reshape(TAIL128, 128)
    rm128 = _sc_transpose_fn()(emb_t, tail128)
    emb_rm = rm128.reshape(TAB_ROWS, D)

    gat_emb, lin_sum = _sc_gather_fn()(idx, idx_hi, emb_rm, lin16)
    emb2d = gat_emb.reshape(B, FD)

    jfd = jnp.arange(FD, dtype=jnp.int32)
    jf = jnp.arange(F, dtype=jnp.int32)
    jd = jnp.arange(D, dtype=jnp.int32)
    e_mat = (jfd[None, :] // D == jf[:, None]).astype(BF)
    et_mat = (jfd[:, None] // D == jf[None, :]).astype(BF)
    p_mat = (jfd[:, None] % D == jd[None, :]).astype(BF)

    full = lambda shape: pl.BlockSpec(shape, lambda i: tuple(0 for _ in shape))
    _, _, sc416, sh416 = pl.pallas_call(
        _stats_kernel,
        grid=(B // BB1,),
        in_specs=[pl.BlockSpec((BB1, FD), lambda i: (i, 0)),
                  full((FD, F)), full((F, FD)), full((1, F)), full((1, F))],
        out_specs=[
            pl.BlockSpec((1, FD), lambda i: (0, 0)),
            pl.BlockSpec((1, FD), lambda i: (0, 0)),
            pl.BlockSpec((1, FD), lambda i: (0, 0)),
            pl.BlockSpec((1, FD), lambda i: (0, 0)),
        ],
        out_shape=[
            jax.ShapeDtypeStruct((1, FD), jnp.float32),
            jax.ShapeDtypeStruct((1, FD), jnp.float32),
            jax.ShapeDtypeStruct((1, FD), jnp.float32),
            jax.ShapeDtypeStruct((1, FD), jnp.float32),
        ],
    )(emb2d, et_mat, e_mat, bn_gamma.reshape(1, F), bn_beta.reshape(1, F))

    out = pl.pallas_call(
        _main_kernel,
        grid=(B // BB2,),
        in_specs=[
            pl.BlockSpec((BB2, FD), lambda i: (i, 0)),
            pl.BlockSpec((BB2,), lambda i: (i,)),
            full((1, FD)),
            full((1, FD)),
            full((F, FD)),
            full((FD, D)),
            full((FD, F)),
            full((1, F)),
            full((1, 1)),
        ],
        out_specs=pl.BlockSpec((BB2,), lambda i: (i,)),
        out_shape=jax.ShapeDtypeStruct((B,), jnp.float32),
    )(emb2d, lin_sum, sc416, sh416, e_mat, p_mat, ctrl_w,
      ctrl_b.reshape(1, F), lin_b.reshape(1, 1))
    return out
